# Initial kernel scaffold; baseline (speedup 1.0000x reference)
import functools

import jax
import jax.numpy as jnp
from jax import lax
from jax.experimental import pallas as pl
from jax.experimental.pallas import tpu as pltpu
from jax.experimental.pallas import tpu_sc as plsc

_C = 128
_NW = 32


def _make_sc_norms(d, ng, arity):
    nchunks = ng // _C
    cpw = nchunks // _NW
    mesh = plsc.VectorSubcoreMesh(core_axis_name="c", subcore_axis_name="s")

    @functools.partial(
        pl.kernel,
        mesh=mesh,
        out_type=jax.ShapeDtypeStruct((ng,), jnp.float32),
        scratch_types=[
            pltpu.VMEM((arity, _C), jnp.int32),
            pltpu.VMEM((_C, d), jnp.float32),
            pltpu.VMEM((_C,), jnp.float32),
            pltpu.SemaphoreType.DMA,
        ],
    )
    def sc_norms(emb_hbm, idx_hbm, out_hbm, idx_v, acc_v, norms_v, sem):
        wid = lax.axis_index("s") * 2 + lax.axis_index("c")

        def chunk_body(ci, carry):
            gci = wid * cpw + ci
            pltpu.sync_copy(idx_hbm.at[gci], idx_v)
            pltpu.async_copy(emb_hbm.at[idx_v.at[0]], acc_v, sem).wait()
            cps = [
                pltpu.async_copy(emb_hbm.at[idx_v.at[k]], acc_v, sem, add=True)
                for k in range(1, arity)
            ]
            for cp in cps:
                cp.wait()

            def grp_body(j, carry2):
                v = acc_v[j, pl.ds(0, 16)]
                s = v * v
                for c in range(1, d // 16):
                    v = acc_v[j, pl.ds(c * 16, 16)]
                    s = s + v * v
                norms_v[j] = jnp.sum(s)
                return carry2

            lax.fori_loop(0, _C, grp_body, 0, unroll=False)
            pltpu.sync_copy(norms_v, out_hbm.at[pl.ds(gci * _C, _C)])
            return carry

        lax.fori_loop(0, cpw, chunk_body, 0, unroll=False)

    return sc_norms


def _make_score(nchunks, batch):
    rows_pos = batch // _C

    def score_body(norms_ref, out_ref):
        x = norms_ref[...]
        rows = lax.broadcasted_iota(jnp.int32, (nchunks, _C), 0)
        v = jnp.where(rows < rows_pos, x, 1.0 / x)
        out_ref[0, 0] = jnp.sum(jnp.log(jnp.tanh(v))) / batch

    return pl.pallas_call(
        score_body,
        out_shape=jax.ShapeDtypeStruct((1, 1), jnp.float32),
        out_specs=pl.BlockSpec(memory_space=pltpu.SMEM),
    )


def kernel(x_pos, x_neg, emb):
    batch, arity = x_pos.shape
    num_neg = x_neg.shape[1]
    d = emb.shape[1]
    ng = batch * (1 + num_neg)
    assert ng % (_C * _NW) == 0 and d % 16 == 0

    groups = jnp.concatenate(
        [x_pos, x_neg.reshape(batch * num_neg, arity)], axis=0
    )
    idx_staged = jnp.transpose(
        groups.reshape(ng // _C, _C, arity), (0, 2, 1)
    )

    norms = _make_sc_norms(d, ng, arity)(emb, idx_staged)
    score = _make_score(ng // _C, batch)(norms.reshape(ng // _C, _C))
    return score[0, 0]


# speedup vs baseline: 2.5050x; 2.5050x over previous
import functools

import jax
import jax.numpy as jnp
from jax import lax
from jax.experimental import pallas as pl
from jax.experimental.pallas import tpu as pltpu
from jax.experimental.pallas import tpu_sc as plsc

_C = 128
_NW = 32


def _make_sc_norms(d, ng, arity):
    nchunks = ng // _C
    cpw = nchunks // _NW
    mesh = plsc.VectorSubcoreMesh(core_axis_name="c", subcore_axis_name="s")

    @functools.partial(
        pl.kernel,
        mesh=mesh,
        compiler_params=pltpu.CompilerParams(use_tc_tiling_on_sc=False),
        out_type=jax.ShapeDtypeStruct((ng, 16), jnp.float32),
        scratch_types=[
            pltpu.VMEM((arity, _C), jnp.int32),
            pltpu.VMEM((_C, d), jnp.float32),
            pltpu.VMEM((_C, 16), jnp.float32),
            pltpu.SemaphoreType.DMA,
        ],
    )
    def sc_norms(emb_hbm, idx_hbm, out_hbm, idx_v, acc_v, norms_v, sem):
        wid = lax.axis_index("s") * 2 + lax.axis_index("c")

        def chunk_body(ci, carry):
            gci = wid * cpw + ci
            pltpu.sync_copy(idx_hbm.at[gci], idx_v)
            pltpu.async_copy(emb_hbm.at[idx_v.at[0]], acc_v, sem).wait()
            cps = [
                pltpu.async_copy(emb_hbm.at[idx_v.at[k]], acc_v, sem, add=True)
                for k in range(1, arity)
            ]
            for cp in cps:
                cp.wait()

            def grp_body(j, carry2):
                v = acc_v[j, pl.ds(0, 16)]
                s = v * v
                for c in range(1, d // 16):
                    v = acc_v[j, pl.ds(c * 16, 16)]
                    s = s + v * v
                norms_v[j, pl.ds(0, 16)] = s
                return carry2

            lax.fori_loop(0, _C, grp_body, 0, unroll=False)
            pltpu.sync_copy(norms_v, out_hbm.at[pl.ds(gci * _C, _C), :])
            return carry

        lax.fori_loop(0, cpw, chunk_body, 0, unroll=False)

    return sc_norms


def _make_score(ng, batch):
    nrows = ng * 16 // 128
    rows_pos = batch // 8

    def score_body(part_ref, out_ref):
        x = part_ref[...]
        l = lax.broadcasted_iota(jnp.int32, (128, 8), 0)
        k = lax.broadcasted_iota(jnp.int32, (128, 8), 1)
        m = (l // 16 == k).astype(jnp.float32)
        y = jnp.dot(x, m, precision=lax.Precision.HIGHEST)
        rows = lax.broadcasted_iota(jnp.int32, (nrows, 8), 0)
        v = jnp.where(rows < rows_pos, y, 1.0 / y)
        out_ref[0, 0] = jnp.sum(jnp.log(jnp.tanh(v))) / batch

    return pl.pallas_call(
        score_body,
        out_shape=jax.ShapeDtypeStruct((1, 1), jnp.float32),
        out_specs=pl.BlockSpec(memory_space=pltpu.SMEM),
    )


def kernel(x_pos, x_neg, emb):
    batch, arity = x_pos.shape
    num_neg = x_neg.shape[1]
    d = emb.shape[1]
    ng = batch * (1 + num_neg)
    assert ng % (_C * _NW) == 0 and d % 16 == 0

    groups = jnp.concatenate(
        [x_pos, x_neg.reshape(batch * num_neg, arity)], axis=0
    )
    idx_staged = jnp.transpose(
        groups.reshape(ng // _C, _C, arity), (0, 2, 1)
    )

    part = _make_sc_norms(d, ng, arity)(emb, idx_staged)
    score = _make_score(ng, batch)(part.reshape(ng * 16 // 128, 128))
    return score[0, 0]


# speedup vs baseline: 2.6425x; 1.0549x over previous
import functools

import jax
import jax.numpy as jnp
from jax import lax
from jax.experimental import pallas as pl
from jax.experimental.pallas import tpu as pltpu
from jax.experimental.pallas import tpu_sc as plsc

_C = 128
_NW = 32


def _make_sc_norms(d, ng, arity):
    nchunks = ng // _C
    cpw = nchunks // _NW
    mesh = plsc.VectorSubcoreMesh(core_axis_name="c", subcore_axis_name="s")

    @functools.partial(
        pl.kernel,
        mesh=mesh,
        compiler_params=pltpu.CompilerParams(use_tc_tiling_on_sc=False),
        out_type=jax.ShapeDtypeStruct((ng, 16), jnp.float32),
        scratch_types=[
            pltpu.VMEM((2, arity, _C), jnp.int32),
            pltpu.VMEM((2, _C, d), jnp.float32),
            pltpu.VMEM((_C, 16), jnp.float32),
            pltpu.SemaphoreType.DMA,
            pltpu.SemaphoreType.DMA,
        ],
    )
    def sc_norms(emb_hbm, idx_hbm, out_hbm, idx_v, acc_v, norms_v, sem0, sem1):
        wid = lax.axis_index("s") * 2 + lax.axis_index("c")
        sems = (sem0, sem1)
        zvec = jnp.zeros((16,), jnp.float32)

        def stage_fire(ci, b, sem):
            pltpu.sync_copy(idx_hbm.at[wid * cpw + ci], idx_v.at[b])
            for k in range(arity):
                pltpu.async_copy(emb_hbm.at[idx_v.at[b, k]], acc_v.at[b], sem, add=True)

        def drain(b, sem):
            for _ in range(arity):
                pltpu.make_async_copy(
                    emb_hbm.at[idx_v.at[b, 0]], acc_v.at[b], sem
                ).wait()

        def compute_out(ci, b):
            def grp_body(j, carry2):
                v = acc_v[b, j, pl.ds(0, 16)]
                s = v * v
                acc_v[b, j, pl.ds(0, 16)] = zvec
                for c in range(1, d // 16):
                    v = acc_v[b, j, pl.ds(c * 16, 16)]
                    s = s + v * v
                    acc_v[b, j, pl.ds(c * 16, 16)] = zvec
                norms_v[j, pl.ds(0, 16)] = s
                return carry2

            lax.fori_loop(0, _C, grp_body, 0, unroll=False)
            pltpu.sync_copy(norms_v, out_hbm.at[pl.ds((wid * cpw + ci) * _C, _C), :])

        def zero_body(j, carry2):
            for b in range(2):
                for c in range(d // 16):
                    acc_v[b, j, pl.ds(c * 16, 16)] = zvec
            return carry2

        lax.fori_loop(0, _C, zero_body, 0, unroll=False)

        stage_fire(0, 0, sem0)

        def pipe_body(h, carry):
            c0 = 2 * h
            stage_fire(c0 + 1, 1, sem1)
            drain(0, sem0)
            compute_out(c0, 0)

            @pl.when(c0 + 2 < cpw)
            def _():
                stage_fire(c0 + 2, 0, sem0)

            drain(1, sem1)
            compute_out(c0 + 1, 1)
            return carry

        lax.fori_loop(0, cpw // 2, pipe_body, 0, unroll=False)

    return sc_norms


def _make_score(ng, batch):
    nrows = ng * 16 // 128
    rows_pos = batch // 8

    def score_body(part_ref, out_ref):
        x = part_ref[...]
        l = lax.broadcasted_iota(jnp.int32, (128, 8), 0)
        k = lax.broadcasted_iota(jnp.int32, (128, 8), 1)
        m = (l // 16 == k).astype(jnp.float32)
        y = jnp.dot(x, m, precision=lax.Precision.HIGHEST)
        rows = lax.broadcasted_iota(jnp.int32, (nrows, 8), 0)
        v = jnp.where(rows < rows_pos, y, 1.0 / y)
        out_ref[0, 0] = jnp.sum(jnp.log(jnp.tanh(v))) / batch

    return pl.pallas_call(
        score_body,
        out_shape=jax.ShapeDtypeStruct((1, 1), jnp.float32),
        out_specs=pl.BlockSpec(memory_space=pltpu.SMEM),
    )


def kernel(x_pos, x_neg, emb):
    batch, arity = x_pos.shape
    num_neg = x_neg.shape[1]
    d = emb.shape[1]
    ng = batch * (1 + num_neg)
    assert ng % (_C * _NW) == 0 and d % 16 == 0

    groups = jnp.concatenate(
        [x_pos, x_neg.reshape(batch * num_neg, arity)], axis=0
    )
    idx_staged = jnp.transpose(
        groups.reshape(ng // _C, _C, arity), (0, 2, 1)
    )

    part = _make_sc_norms(d, ng, arity)(emb, idx_staged)
    score = _make_score(ng, batch)(part.reshape(ng * 16 // 128, 128))
    return score[0, 0]
